# baseline (speedup 1.0000x reference)
import functools

import jax
import jax.numpy as jnp
from jax import lax
from jax.experimental import pallas as pl
from jax.experimental.pallas import tpu as pltpu
from jax.experimental.pallas import tpu_sc as plsc

N_EXP = 256
TOPK = 6
SCALE = 1.5
TBLK = 256
N_TOKENS = 8192
NW = 32
TPW = N_TOKENS // NW
L = 16


def _score_block(x_ref, w_ref, b_ref, out_ref):
    scores = lax.dot_general(
        w_ref[...], x_ref[...],
        (((1,), (1,)), ((), ())),
        preferred_element_type=jnp.float32,
    )
    out_ref[...] = jnp.sqrt(jax.nn.softplus(scores)) + b_ref[...].reshape(N_EXP, 1)


def _insert(vs, ids, newv, newi):
    vs = list(vs)
    ids = list(ids)
    for j in range(TOPK):
        gt = newv > vs[j]
        vj = jnp.where(gt, newv, vs[j])
        ij = jnp.where(gt, newi, ids[j])
        newv = jnp.where(gt, vs[j], newv)
        newi = jnp.where(gt, ids[j], newi)
        vs[j] = vj
        ids[j] = ij
    return tuple(vs), tuple(ids)


TCHUNK = 128


def _topk_sc(bsc_hbm, bias_hbm, w_hbm, i_hbm, bs_v, bias_v, wout_v, iout_v):
    wid = lax.axis_index("s") * 2 + lax.axis_index("c")
    base = wid * TPW
    pltpu.sync_copy(bias_hbm, bias_v)

    lane = lax.broadcasted_iota(jnp.int32, (L,), 0)
    neg = jnp.full((L,), -jnp.inf, jnp.float32)
    zero = jnp.zeros((L,), jnp.int32)

    for c in range(TPW // TCHUNK):
        pltpu.sync_copy(bsc_hbm.at[:, pl.ds(base + c * TCHUNK, TCHUNK)], bs_v)
        for half in range(TCHUNK // (2 * L)):
            g0 = 2 * half
            g1 = g0 + 1

            def body(e, carry, g0=g0, g1=g1):
                (v0, i0, v1, i1) = carry
                sv0 = bs_v[e, pl.ds(g0 * L, L)]
                sv1 = bs_v[e, pl.ds(g1 * L, L)]
                ei = jnp.full((L,), e, jnp.int32)
                v0, i0 = _insert(v0, i0, sv0, ei)
                v1, i1 = _insert(v1, i1, sv1, ei)
                return (v0, i0, v1, i1)

            init = ((neg,) * TOPK, (zero,) * TOPK, (neg,) * TOPK, (zero,) * TOPK)
            v0, i0, v1, i1 = lax.fori_loop(0, N_EXP, body, init)

            for g, vs, ids in ((g0, v0, i0), (g1, v1, i1)):
                ws = []
                for j in range(TOPK):
                    bj = plsc.load_gather(bias_v, [ids[j]])
                    ws.append(vs[j] - bj)
                tot = ws[0] + ws[1] + ws[2] + ws[3] + ws[4] + ws[5]
                inv = SCALE / tot
                rows = c * TCHUNK + g * L + lane
                for j in range(TOPK):
                    col = jnp.full((L,), j, jnp.int32)
                    plsc.store_scatter(wout_v, [rows, col], ws[j] * inv)
                    plsc.store_scatter(iout_v, [rows, col], ids[j])

    pltpu.sync_copy(wout_v, w_hbm.at[pl.ds(base, TPW)])
    pltpu.sync_copy(iout_v, i_hbm.at[pl.ds(base, TPW)])


@jax.jit
def kernel(x, W, bias):
    n_tokens = x.shape[0]
    bsc = pl.pallas_call(
        _score_block,
        grid=(n_tokens // TBLK,),
        in_specs=[
            pl.BlockSpec((TBLK, x.shape[1]), lambda i: (i, 0)),
            pl.BlockSpec((N_EXP, x.shape[1]), lambda i: (0, 0)),
            pl.BlockSpec((N_EXP,), lambda i: (0,)),
        ],
        out_specs=pl.BlockSpec((N_EXP, TBLK), lambda i: (0, i)),
        out_shape=jax.ShapeDtypeStruct((N_EXP, n_tokens), jnp.float32),
    )(x, W, bias)

    mesh = plsc.VectorSubcoreMesh(core_axis_name="c", subcore_axis_name="s")
    topk = functools.partial(
        pl.kernel,
        mesh=mesh,
        out_type=[
            jax.ShapeDtypeStruct((n_tokens, TOPK), jnp.float32),
            jax.ShapeDtypeStruct((n_tokens, TOPK), jnp.int32),
        ],
        scratch_types=[
            pltpu.VMEM((N_EXP, TCHUNK), jnp.float32),
            pltpu.VMEM((N_EXP,), jnp.float32),
            pltpu.VMEM((TPW, TOPK), jnp.float32),
            pltpu.VMEM((TPW, TOPK), jnp.int32),
        ],
        compiler_params=pltpu.CompilerParams(
            needs_layout_passes=False, use_tc_tiling_on_sc=False),
    )(_topk_sc)
    wout, iout = topk(bsc, bias)
    return (wout, iout)


# speedup vs baseline: 1.0765x; 1.0765x over previous
import functools

import jax
import jax.numpy as jnp
from jax import lax
from jax.experimental import pallas as pl
from jax.experimental.pallas import tpu as pltpu
from jax.experimental.pallas import tpu_sc as plsc

N_EXP = 256
TOPK = 6
SCALE = 1.5
TBLK = 256
NW = 32
L = 16
CHUNKS = 4


def _score_block(x_ref, w_ref, b_ref, out_ref):
    scores = lax.dot_general(
        w_ref[...], x_ref[...],
        (((1,), (1,)), ((), ())),
        preferred_element_type=jnp.float32,
    )
    out_ref[...] = jnp.sqrt(jax.nn.softplus(scores)) + b_ref[...].reshape(N_EXP, 1)


def _insert(vs, ids, newv, newi):
    vs = list(vs)
    ids = list(ids)
    for j in range(TOPK):
        gt = newv > vs[j]
        vj = jnp.where(gt, newv, vs[j])
        ij = jnp.where(gt, newi, ids[j])
        newv = jnp.where(gt, vs[j], newv)
        newi = jnp.where(gt, ids[j], newi)
        vs[j] = vj
        ids[j] = ij
    return tuple(vs), tuple(ids)


def _make_topk_sc(tpw):
    def _topk_sc(bsc_hbm, bias_hbm, w_hbm, i_hbm, bs_v, bias_v, wout_v, iout_v):
        wid = lax.axis_index("s") * 2 + lax.axis_index("c")
        base = wid * tpw
        pltpu.sync_copy(bias_hbm, bias_v)
        pltpu.sync_copy(bsc_hbm.at[:, pl.ds(base, tpw)], bs_v)

        lane = lax.broadcasted_iota(jnp.int32, (L,), 0)
        neg = jnp.full((L,), -jnp.inf, jnp.float32)
        zero = jnp.zeros((L,), jnp.int32)

        for half in range(tpw // (2 * L)):
            g0 = 2 * half
            g1 = g0 + 1

            def body(e, carry, g0=g0, g1=g1):
                (v0, i0, v1, i1) = carry
                sv0 = bs_v[e, pl.ds(g0 * L, L)]
                sv1 = bs_v[e, pl.ds(g1 * L, L)]
                ei = jnp.full((L,), e, jnp.int32)
                v0, i0 = _insert(v0, i0, sv0, ei)
                v1, i1 = _insert(v1, i1, sv1, ei)
                return (v0, i0, v1, i1)

            init = ((neg,) * TOPK, (zero,) * TOPK, (neg,) * TOPK, (zero,) * TOPK)
            v0, i0, v1, i1 = lax.fori_loop(0, N_EXP, body, init)

            for g, vs, ids in ((g0, v0, i0), (g1, v1, i1)):
                ws = []
                for j in range(TOPK):
                    bj = plsc.load_gather(bias_v, [ids[j]])
                    ws.append(vs[j] - bj)
                tot = ws[0] + ws[1] + ws[2] + ws[3] + ws[4] + ws[5]
                inv = SCALE / tot
                rows = g * L + lane
                for j in range(TOPK):
                    col = jnp.full((L,), j, jnp.int32)
                    plsc.store_scatter(wout_v, [rows, col], ws[j] * inv)
                    plsc.store_scatter(iout_v, [rows, col], ids[j])

        pltpu.sync_copy(wout_v, w_hbm.at[pl.ds(base, tpw)])
        pltpu.sync_copy(iout_v, i_hbm.at[pl.ds(base, tpw)])

    return _topk_sc


@jax.jit
def kernel(x, W, bias):
    n_tokens = x.shape[0]
    ctok = n_tokens // CHUNKS
    tpw = ctok // NW
    blocks_per_chunk = ctok // TBLK

    mesh = plsc.VectorSubcoreMesh(core_axis_name="c", subcore_axis_name="s")
    topk = functools.partial(
        pl.kernel,
        mesh=mesh,
        out_type=[
            jax.ShapeDtypeStruct((ctok, TOPK), jnp.float32),
            jax.ShapeDtypeStruct((ctok, TOPK), jnp.int32),
        ],
        scratch_types=[
            pltpu.VMEM((N_EXP, tpw), jnp.float32),
            pltpu.VMEM((N_EXP,), jnp.float32),
            pltpu.VMEM((tpw, TOPK), jnp.float32),
            pltpu.VMEM((tpw, TOPK), jnp.int32),
        ],
        compiler_params=pltpu.CompilerParams(
            needs_layout_passes=False, use_tc_tiling_on_sc=False),
    )(_make_topk_sc(tpw))

    w_parts = []
    i_parts = []
    for c in range(CHUNKS):
        bsc = pl.pallas_call(
            _score_block,
            grid=(blocks_per_chunk,),
            in_specs=[
                pl.BlockSpec((TBLK, x.shape[1]),
                             lambda i, c=c: (c * blocks_per_chunk + i, 0)),
                pl.BlockSpec((N_EXP, x.shape[1]), lambda i: (0, 0)),
                pl.BlockSpec((N_EXP,), lambda i: (0,)),
            ],
            out_specs=pl.BlockSpec((N_EXP, TBLK), lambda i: (0, i)),
            out_shape=jax.ShapeDtypeStruct((N_EXP, ctok), jnp.float32),
        )(x, W, bias)
        wc, ic = topk(bsc, bias)
        w_parts.append(wc)
        i_parts.append(ic)

    return (jnp.concatenate(w_parts, 0), jnp.concatenate(i_parts, 0))


# speedup vs baseline: 1.1253x; 1.0453x over previous
import functools

import jax
import jax.numpy as jnp
from jax import lax
from jax.experimental import pallas as pl
from jax.experimental.pallas import tpu as pltpu
from jax.experimental.pallas import tpu_sc as plsc

N_EXP = 256
TOPK = 6
SCALE = 1.5
TBLK = 256
NW = 32
L = 16
CHUNKS = 2
UNROLL = 4


def _score_block(x_ref, w_ref, b_ref, out_ref):
    scores = lax.dot_general(
        w_ref[...], x_ref[...],
        (((1,), (1,)), ((), ())),
        preferred_element_type=jnp.float32,
    )
    out_ref[...] = jnp.sqrt(jax.nn.softplus(scores)) + b_ref[...].reshape(N_EXP, 1)


def _insert(vs, ids, newv, newi):
    vs = list(vs)
    ids = list(ids)
    for j in range(TOPK):
        gt = newv > vs[j]
        vj = jnp.where(gt, newv, vs[j])
        ij = jnp.where(gt, newi, ids[j])
        newv = jnp.where(gt, vs[j], newv)
        newi = jnp.where(gt, ids[j], newi)
        vs[j] = vj
        ids[j] = ij
    return tuple(vs), tuple(ids)


def _make_topk_sc(tpw):
    def _topk_sc(bsc_hbm, bias_hbm, w_hbm, i_hbm, bs_v, bias_v, wout_v, iout_v):
        wid = lax.axis_index("s") * 2 + lax.axis_index("c")
        base = wid * tpw
        pltpu.sync_copy(bias_hbm, bias_v)
        pltpu.sync_copy(bsc_hbm.at[:, pl.ds(base, tpw)], bs_v)

        lane = lax.broadcasted_iota(jnp.int32, (L,), 0)
        neg = jnp.full((L,), -jnp.inf, jnp.float32)
        zero = jnp.zeros((L,), jnp.int32)

        for half in range(tpw // (2 * L)):
            g0 = 2 * half
            g1 = g0 + 1

            def body(e, carry, g0=g0, g1=g1):
                (v0, i0, v1, i1) = carry
                sv0 = bs_v[e, pl.ds(g0 * L, L)]
                sv1 = bs_v[e, pl.ds(g1 * L, L)]
                ei = jnp.full((L,), e, jnp.int32)
                v0, i0 = _insert(v0, i0, sv0, ei)
                v1, i1 = _insert(v1, i1, sv1, ei)
                return (v0, i0, v1, i1)

            init = ((neg,) * TOPK, (zero,) * TOPK, (neg,) * TOPK, (zero,) * TOPK)
            v0, i0, v1, i1 = lax.fori_loop(0, N_EXP, body, init, unroll=UNROLL)

            for g, vs, ids in ((g0, v0, i0), (g1, v1, i1)):
                ws = []
                for j in range(TOPK):
                    bj = plsc.load_gather(bias_v, [ids[j]])
                    ws.append(vs[j] - bj)
                tot = ws[0] + ws[1] + ws[2] + ws[3] + ws[4] + ws[5]
                inv = SCALE / tot
                rows = g * L + lane
                for j in range(TOPK):
                    col = jnp.full((L,), j, jnp.int32)
                    plsc.store_scatter(wout_v, [rows, col], ws[j] * inv)
                    plsc.store_scatter(iout_v, [rows, col], ids[j])

        pltpu.sync_copy(wout_v, w_hbm.at[pl.ds(base, tpw)])
        pltpu.sync_copy(iout_v, i_hbm.at[pl.ds(base, tpw)])

    return _topk_sc


@jax.jit
def kernel(x, W, bias):
    n_tokens = x.shape[0]
    ctok = n_tokens // CHUNKS
    tpw = ctok // NW
    blocks_per_chunk = ctok // TBLK

    mesh = plsc.VectorSubcoreMesh(core_axis_name="c", subcore_axis_name="s")
    topk = functools.partial(
        pl.kernel,
        mesh=mesh,
        out_type=[
            jax.ShapeDtypeStruct((ctok, TOPK), jnp.float32),
            jax.ShapeDtypeStruct((ctok, TOPK), jnp.int32),
        ],
        scratch_types=[
            pltpu.VMEM((N_EXP, tpw), jnp.float32),
            pltpu.VMEM((N_EXP,), jnp.float32),
            pltpu.VMEM((tpw, TOPK), jnp.float32),
            pltpu.VMEM((tpw, TOPK), jnp.int32),
        ],
        compiler_params=pltpu.CompilerParams(
            needs_layout_passes=False, use_tc_tiling_on_sc=False),
    )(_make_topk_sc(tpw))

    w_parts = []
    i_parts = []
    for c in range(CHUNKS):
        bsc = pl.pallas_call(
            _score_block,
            grid=(blocks_per_chunk,),
            in_specs=[
                pl.BlockSpec((TBLK, x.shape[1]),
                             lambda i, c=c: (c * blocks_per_chunk + i, 0)),
                pl.BlockSpec((N_EXP, x.shape[1]), lambda i: (0, 0)),
                pl.BlockSpec((N_EXP,), lambda i: (0,)),
            ],
            out_specs=pl.BlockSpec((N_EXP, TBLK), lambda i: (0, i)),
            out_shape=jax.ShapeDtypeStruct((N_EXP, ctok), jnp.float32),
        )(x, W, bias)
        wc, ic = topk(bsc, bias)
        w_parts.append(wc)
        i_parts.append(ic)

    return (jnp.concatenate(w_parts, 0), jnp.concatenate(i_parts, 0))
